# Initial kernel scaffold; baseline (speedup 1.0000x reference)
import jax
import jax.numpy as jnp
from jax.experimental import pallas as pl


def kernel(x, edge_index, templates_features, templates):
    raise NotImplementedError("write your pallas kernel here")


# speedup vs baseline: 3.1565x; 3.1565x over previous
import functools

import jax
import jax.numpy as jnp
from jax import lax
from jax.experimental import pallas as pl
from jax.experimental.pallas import tpu as pltpu
from jax.experimental.pallas import tpu_sc as plsc

N = 10000
NPAD = 10240
E = 320000
D = 128
DW = D
T = 10
K = 5
TK = T * K

NC = 2
NS = 16
NW = NC * NS
EPW = E // NW
C = 80
NCHUNK = EPW // C
ROWS_PER_TILE = NPAD // NS

ALPHA = 0.5
REG = 0.1
N_ITER = 30

def _sc_agg_body(src_hbm, dst_hbm, x_hbm, agg_out, deg_out,
                 src_v, dst_v, rows_v, zrows_v, ones_v, acc_sh, sem):
    c = lax.axis_index("c")
    s = lax.axis_index("s")
    wid = s * NC + c
    r0 = s * ROWS_PER_TILE

    zero16 = jnp.zeros((16,), jnp.float32)
    one16 = jnp.ones((16,), jnp.float32)
    for i in range(C):
        for j in range(D // 16):
            zrows_v[i, pl.ds(j * 16, 16)] = zero16
            ones_v[i, pl.ds(j * 16, 16)] = one16

    for b in range(ROWS_PER_TILE // C):
        pltpu.sync_copy(zrows_v, acc_sh.at[pl.ds(r0 + b * C, C)])
    plsc.subcore_barrier()

    def body1(i, carry):
        base = wid * EPW + i * C
        pltpu.sync_copy(src_hbm.at[pl.ds(base, C)], src_v)
        pltpu.sync_copy(dst_hbm.at[pl.ds(base, C)], dst_v)
        pltpu.async_copy(x_hbm.at[src_v], rows_v, sem).wait()
        pltpu.sync_copy(rows_v, acc_sh.at[dst_v], add=True)
        return carry

    lax.fori_loop(0, NCHUNK, body1, 0)
    plsc.subcore_barrier()

    for b in range(ROWS_PER_TILE // C):
        r = r0 + b * C
        pltpu.sync_copy(acc_sh.at[pl.ds(r, C)], rows_v)
        pltpu.sync_copy(rows_v, agg_out.at[c, pl.ds(r, C)])
        pltpu.sync_copy(zrows_v, acc_sh.at[pl.ds(r, C)])
    plsc.subcore_barrier()

    def body2(i, carry):
        base = wid * EPW + i * C
        pltpu.sync_copy(dst_hbm.at[pl.ds(base, C)], dst_v)
        pltpu.sync_copy(ones_v, acc_sh.at[dst_v], add=True)
        return carry

    lax.fori_loop(0, NCHUNK, body2, 0)
    plsc.subcore_barrier()

    for b in range(ROWS_PER_TILE // C):
        r = r0 + b * C
        pltpu.sync_copy(acc_sh.at[pl.ds(r, C)], rows_v)
        pltpu.sync_copy(rows_v, deg_out.at[c, pl.ds(r, C)])


@functools.cache
def _sc_agg_kernel():
    mesh = plsc.VectorSubcoreMesh(core_axis_name="c", subcore_axis_name="s",
                                  num_cores=NC, num_subcores=NS)
    return pl.kernel(
        _sc_agg_body,
        out_type=(
            jax.ShapeDtypeStruct((NC, NPAD, D), jnp.float32),
            jax.ShapeDtypeStruct((NC, NPAD, D), jnp.float32),
        ),
        mesh=mesh,
        scratch_types=(
            pltpu.VMEM((C,), jnp.int32),
            pltpu.VMEM((C,), jnp.int32),
            pltpu.VMEM((C, D), jnp.float32),
            pltpu.VMEM((C, D), jnp.float32),
            pltpu.VMEM((C, D), jnp.float32),
            pltpu.VMEM_SHARED((NPAD, D), jnp.float32),
            pltpu.SemaphoreType.DMA,
        ),
    )


BN = 512
GRID = NPAD // BN

_DN_RT = (((1,), (1,)), ((), ()))


def _tc_body(x_ref, agg_ref, deg_ref, f_ref, s_ref, o_ref):
    x_blk = x_ref[...]
    agg_blk = agg_ref[0] + agg_ref[1]
    deg_col = deg_ref[0, :, 0:1] + deg_ref[1, :, 0:1]
    neigh = agg_blk / jnp.maximum(deg_col, 1.0)
    F = f_ref[...]
    S_all = s_ref[...]

    Fsq = jnp.sum(F * F, axis=1, keepdims=True)
    dxT = lax.dot_general(F, x_blk, _DN_RT,
                          preferred_element_type=jnp.float32)
    dnT = lax.dot_general(F, neigh, _DN_RT,
                          preferred_element_type=jnp.float32)
    one_row = jnp.full((1, D), 1.0, jnp.float32)
    xsqT = lax.dot_general(one_row, x_blk * x_blk, _DN_RT,
                           preferred_element_type=jnp.float32)
    nsqT = lax.dot_general(one_row, neigh * neigh, _DN_RT,
                           preferred_element_type=jnp.float32)

    Mx = jnp.maximum(xsqT + Fsq - 2.0 * dxT, 0.0)
    Mn = jnp.maximum(nsqT + Fsq - 2.0 * dnT, 0.0)
    Kx = jnp.exp(Mx * (-1.0 / REG))
    Kn = jnp.exp(Mn * (-1.0 / REG))

    outs = []
    for t in range(T):
        sl = slice(K * t, K * t + K)
        Kxt = Kx[sl]
        Knt = Kn[sl]
        Mxt = Mx[sl]
        Mnt = Mn[sl]

        def body(i, carry, Kxt=Kxt, Knt=Knt):
            u0, u1, v = carry
            c0 = jnp.sum(Kxt * v, axis=0, keepdims=True)
            c1 = jnp.sum(Knt * v, axis=0, keepdims=True)
            u0 = 0.5 / (c0 + 1e-16)
            u1 = 0.5 / (c1 + 1e-16)
            d = Kxt * u0 + Knt * u1
            v = (1.0 / K) / (d + 1e-16)
            return (u0, u1, v)

        init = (jnp.full((1, BN), 0.5, jnp.float32),
                jnp.full((1, BN), 0.5, jnp.float32),
                jnp.ones((K, BN), jnp.float32))
        u0, u1, v = lax.fori_loop(0, N_ITER, body, init)

        P0 = u0 * Kxt * v
        P1 = u1 * Knt * v
        wass = jnp.sum(Mxt * P0 + Mnt * P1, axis=0, keepdims=True)

        St = S_all[t]
        S2t = St * St
        mu0 = jnp.sum(P0, axis=0, keepdims=True)
        mu1 = jnp.sum(P1, axis=0, keepdims=True)
        nu = P0 + P1
        w2 = jnp.sum(S2t[:, :, None] * nu[None, :, :], axis=1)
        w1 = jnp.sum(St[:, :, None] * P1[None, :, :], axis=1)
        w0 = jnp.sum(St[:, :, None] * P0[None, :, :], axis=1)
        term2 = jnp.sum(nu * w2, axis=0, keepdims=True)
        term3 = -2.0 * (jnp.sum(P0 * w1, axis=0, keepdims=True)
                        + jnp.sum(P1 * w0, axis=0, keepdims=True))
        gw = 2.0 * mu0 * mu1 + term2 + term3
        outs.append((1.0 - ALPHA) * wass + ALPHA * gw)

    o_ref[...] = jnp.concatenate(outs, axis=0)


def _tc_call(xpad, agg2, deg2, f_flat, templates, interpret=False):
    return pl.pallas_call(
        _tc_body,
        grid=(GRID,),
        in_specs=[
            pl.BlockSpec((BN, D), lambda i: (i, 0)),
            pl.BlockSpec((NC, BN, D), lambda i: (0, i, 0)),
            pl.BlockSpec((NC, BN, D), lambda i: (0, i, 0)),
            pl.BlockSpec((TK, D), lambda i: (0, 0)),
            pl.BlockSpec((T, K, K), lambda i: (0, 0, 0)),
        ],
        out_specs=pl.BlockSpec((T, BN), lambda i: (0, i)),
        out_shape=jax.ShapeDtypeStruct((T, NPAD), jnp.float32),
        interpret=(pltpu.InterpretParams() if interpret else False),
    )(xpad, agg2, deg2, f_flat, templates)


def kernel(x, edge_index, templates_features, templates):
    src = edge_index[0]
    dst = edge_index[1]
    agg2, deg2 = _sc_agg_kernel()(src, dst, x)
    xpad = jnp.pad(x, ((0, NPAD - N), (0, 0)))
    f_flat = templates_features.reshape(TK, D)
    outT = _tc_call(xpad, agg2, deg2, f_flat, templates)
    return outT[:, :N].T
